# baseline (speedup 1.0000x reference)
import functools

import jax
import jax.numpy as jnp
from jax import lax
from jax.experimental import pallas as pl
from jax.experimental.pallas import tpu as pltpu
from jax.experimental.pallas import tpu_sc as plsc

_NC = 2
_NS = 16
_NW = _NC * _NS

_CH = 128


@functools.cache
def _build(B, V2, D2):
    assert B % (_NW * _CH) == 0
    bpw = B // _NW
    nch = bpw // _CH
    D = D2 // 2

    mesh = plsc.VectorSubcoreMesh(core_axis_name="c", subcore_axis_name="s")

    @functools.partial(
        pl.kernel,
        mesh=mesh,
        out_type=jax.ShapeDtypeStruct((B // 2, D2), jnp.float32),
        scratch_types=[
            pltpu.VMEM((bpw + 16,), jnp.int32),
            pltpu.VMEM((bpw,), jnp.int32),
            pltpu.VMEM((_CH, D2), jnp.float32),
            pltpu.VMEM((_CH, D2), jnp.float32),
            pltpu.VMEM((_CH // 2, D2), jnp.float32),
            pltpu.VMEM((_CH // 2, D2), jnp.float32),
            pltpu.SemaphoreType.DMA,
            pltpu.SemaphoreType.DMA,
        ],
    )
    def gather_kernel(table_hbm, idx_hbm, out_hbm, idx_v, idx2_v,
                      g0, g1, t0, t1, gsem, wsem):
        rows = (g0, g1)
        trs = (t0, t1)
        wid = lax.axis_index("s") * _NC + lax.axis_index("c")
        base = wid * bpw
        pltpu.sync_copy(idx_hbm.at[pl.ds(base, bpw)], idx_v.at[pl.ds(0, bpw)])

        def halve(i, c):
            for u in range(4):
                k = i * 64 + u * 16
                idx2_v[pl.ds(k, 16)] = lax.shift_right_logical(
                    idx_v[pl.ds(k, 16)], 1
                )
            return c

        lax.fori_loop(0, bpw // 64, halve, 0)

        def gather_cp(j, slot):
            return pltpu.make_async_copy(
                table_hbm.at[idx2_v.at[pl.ds(j * _CH, _CH)]],
                rows[slot],
                gsem,
            )

        def write_cp(j, slot):
            off = pl.multiple_of((base + j * _CH) // 2, 8)
            return pltpu.make_async_copy(
                trs[slot],
                out_hbm.at[pl.ds(off, _CH // 2)],
                wsem,
            )

        gather_cp(0, 0).start()
        gather_cp(1, 1).start()

        def body(i, carry):
            for par in range(2):
                j = 2 * i + par
                gather_cp(j, par).wait()

                @pl.when(j >= 2)
                def _():
                    write_cp(j - 2, par).wait()

                g = rows[par]
                t = trs[par]

                def extract(b, c):
                    hv = idx_v[pl.ds(j * _CH + b, 16)]
                    h = (hv[0] & 1) * D
                    for dd in range(D // 16):
                        t[b // 2, pl.ds((b % 2) * D + dd * 16, 16)] = (
                            g[b, pl.ds(h + dd * 16, 16)]
                        )
                    return c

                lax.fori_loop(0, _CH, extract, 0)

                write_cp(j, par).start()

                @pl.when(j + 2 < nch)
                def _():
                    gather_cp(j + 2, par).start()

            return carry

        lax.fori_loop(0, nch // 2, body, 0)

        write_cp(nch - 2, 0).wait()
        write_cp(nch - 1, 1).wait()

    return gather_kernel


def kernel(questions, table):
    Bq, S = questions.shape
    V, D = table.shape
    idx = questions.T.reshape(-1).astype(jnp.int32)
    t2 = table.reshape(V // 2, 2 * D)
    out = _build(Bq * S, V // 2, 2 * D)(t2, idx)
    return out.reshape(S, Bq, D).transpose(1, 0, 2)
